# baseline (speedup 1.0000x reference)
import jax
import jax.numpy as jnp
from jax.experimental import pallas as pl
from jax.experimental.pallas import tpu as pltpu

N_NODES = 10000
HID = 128
NREL = 8
NEDGE = 320000
CHUNK = 512
NCHUNK = NEDGE // CHUNK


CNT_ROWS = NREL * N_NODES // HID


def _count_body(dst_ref, typ_ref, inv_ref, cnt_ref):
    pid = pl.program_id(0)

    @pl.when(pid == 0)
    def _init():
        cnt_ref[...] = jnp.zeros_like(cnt_ref)

    lane_iota = jax.lax.broadcasted_iota(jnp.int32, (1, HID), 1)

    def body(e, carry):
        d = dst_ref[0, 0, e]
        t = typ_ref[0, 0, e]
        idx = t * N_NODES + d
        row = idx // HID
        oh = (lane_iota == idx % HID).astype(jnp.float32)
        cnt_ref[pl.ds(row, 1), :] += oh
        return carry

    jax.lax.fori_loop(0, CHUNK, body, 0, unroll=2)

    @pl.when(pid == NCHUNK - 1)
    def _finish():
        inv_ref[...] = 1.0 / jnp.maximum(cnt_ref[...], 1.0)


def _counts(dst3, typ3):
    return pl.pallas_call(
        _count_body,
        grid=(NCHUNK,),
        in_specs=[
            pl.BlockSpec((1, 1, CHUNK), lambda i: (i, 0, 0),
                         memory_space=pltpu.SMEM),
            pl.BlockSpec((1, 1, CHUNK), lambda i: (i, 0, 0),
                         memory_space=pltpu.SMEM),
        ],
        out_specs=pl.BlockSpec((CNT_ROWS, HID), lambda i: (0, 0)),
        out_shape=jax.ShapeDtypeStruct((CNT_ROWS, HID), jnp.float32),
        scratch_shapes=[pltpu.VMEM((CNT_ROWS, HID), jnp.float32)],
        compiler_params=pltpu.CompilerParams(
            dimension_semantics=("arbitrary",)),
    )(dst3, typ3)


def _make_layer_body(do_relu):
    def _layer_body(src_ref, dst_ref, typ_ref, tv_ref, x_ref, inv_ref,
                    wself_ref, wpart_ref, perm_ref, root_ref, bias_ref,
                    out_ref, xc_ref, sc_ref):
        pid = pl.program_id(0)

        @pl.when(pid == 0)
        def _init():
            out_ref[...] = jnp.dot(
                x_ref[...], root_ref[...],
                preferred_element_type=jnp.float32) + bias_ref[...]

        lane_iota = jax.lax.broadcasted_iota(jnp.int32, (1, HID), 1)

        def gather_body(e, carry):
            s = src_ref[0, 0, e]
            d = dst_ref[0, 0, e]
            t = typ_ref[0, 0, e]
            idx = t * N_NODES + d
            xc_ref[pl.ds(e, 1), :] = x_ref[pl.ds(s, 1), :]
            row = inv_ref[pl.ds(idx // HID, 1), :]
            oh = (lane_iota == idx % HID).astype(jnp.float32)
            sc_ref[pl.ds(e, 1), :] = jnp.sum(row * oh, axis=1, keepdims=True)
            return carry

        jax.lax.fori_loop(0, CHUNK, gather_body, 0, unroll=2)

        xs = xc_ref[...] * sc_ref[...]
        xw = jnp.dot(xs, perm_ref[...],
                     preferred_element_type=jnp.float32)
        tv = tv_ref[0]
        h = jnp.zeros_like(xs)
        for r in range(NREL):
            m = (tv == r).astype(jnp.float32)
            h += m * (xs * wself_ref[r, :][None, :]
                      + xw * wpart_ref[r, :][None, :])
        xc_ref[...] = h

        def scatter_body(e, carry):
            d = dst_ref[0, 0, e]
            out_ref[pl.ds(d, 1), :] += xc_ref[pl.ds(e, 1), :]
            return carry

        jax.lax.fori_loop(0, CHUNK, scatter_body, 0, unroll=2)

        if do_relu:
            @pl.when(pid == NCHUNK - 1)
            def _act():
                out_ref[...] = jnp.maximum(out_ref[...], 0.0)

    return _layer_body


def _layer(src3, dst3, typ3, tv3, x, inv, wself, wpart, perm, root, bias2d,
           do_relu):
    smem = lambda: pl.BlockSpec((1, 1, CHUNK), lambda i: (i, 0, 0),
                                memory_space=pltpu.SMEM)
    full = lambda shape: pl.BlockSpec(shape, lambda i: tuple(0 for _ in shape))
    return pl.pallas_call(
        _make_layer_body(do_relu),
        grid=(NCHUNK,),
        in_specs=[
            smem(), smem(), smem(),
            pl.BlockSpec((1, CHUNK, 1), lambda i: (i, 0, 0)),
            full((N_NODES, HID)),
            full((CNT_ROWS, HID)),
            full((NREL, HID)),
            full((NREL, HID)),
            full((HID, HID)),
            full((HID, HID)),
            full((1, HID)),
        ],
        out_specs=pl.BlockSpec((N_NODES, HID), lambda i: (0, 0)),
        out_shape=jax.ShapeDtypeStruct((N_NODES, HID), jnp.float32),
        scratch_shapes=[
            pltpu.VMEM((CHUNK, HID), jnp.float32),
            pltpu.VMEM((CHUNK, 1), jnp.float32),
        ],
        compiler_params=pltpu.CompilerParams(
            dimension_semantics=("arbitrary",)),
    )(src3, dst3, typ3, tv3, x, inv, wself, wpart, perm, root, bias2d)


def _lane_weights(w):
    wself = jnp.einsum('rbii->rbi', w).reshape(NREL, HID)
    wpart = jnp.einsum('rbii->rbi', w[:, :, ::-1, :]).reshape(NREL, HID)
    return wself, wpart


def kernel(edge_index, edge_type, node_emb, weight1, root1, bias1,
           weight2, root2, bias2):
    src3 = edge_index[0].reshape(NCHUNK, 1, CHUNK)
    dst3 = edge_index[1].reshape(NCHUNK, 1, CHUNK)
    typ3 = edge_type.reshape(NCHUNK, 1, CHUNK)
    tv3 = edge_type.reshape(NCHUNK, CHUNK, 1)

    inv = _counts(dst3, typ3)

    lanes = jnp.arange(HID, dtype=jnp.int32)
    perm = (lanes[:, None] == (lanes[None, :] ^ 1)).astype(jnp.float32).T

    wself1, wpart1 = _lane_weights(weight1)
    wself2, wpart2 = _lane_weights(weight2)

    x1 = _layer(src3, dst3, typ3, tv3, node_emb, inv, wself1, wpart1, perm,
                root1, bias1.reshape(1, HID), do_relu=True)
    x2 = _layer(src3, dst3, typ3, tv3, x1, inv, wself2, wpart2, perm,
                root2, bias2.reshape(1, HID), do_relu=False)
    return x2


# speedup vs baseline: 5.5043x; 5.5043x over previous
import jax
import jax.numpy as jnp
from jax.experimental import pallas as pl
from jax.experimental.pallas import tpu as pltpu

N_NODES = 10000
HID = 128
NREL = 8
NEDGE = 320000
CHUNK = 512
NCHUNK = NEDGE // CHUNK


CNT_ROWS = NREL * N_NODES // HID


NACC = 4


def _count_body(dst_ref, typ_ref, inv_ref, c0, c1, c2, c3):
    pid = pl.program_id(0)
    cnts = (c0, c1, c2, c3)

    @pl.when(pid == 0)
    def _init():
        for c in cnts:
            c[...] = jnp.zeros_like(c)

    lane_iota = jax.lax.broadcasted_iota(jnp.int32, (1, HID), 1)

    def body(i, carry):
        for k in range(NACC):
            e = i * NACC + k
            d = dst_ref[0, 0, e]
            t = typ_ref[0, 0, e]
            idx = t * N_NODES + d
            row = jax.lax.shift_right_logical(idx, 7)
            oh = (lane_iota == jnp.bitwise_and(idx, 127)).astype(jnp.float32)
            cnts[k][pl.ds(row, 1), :] += oh
        return carry

    jax.lax.fori_loop(0, CHUNK // NACC, body, 0)

    @pl.when(pid == NCHUNK - 1)
    def _finish():
        tot = c0[...] + c1[...] + c2[...] + c3[...]
        inv_ref[...] = 1.0 / jnp.maximum(tot, 1.0)


def _counts(dst3, typ3):
    return pl.pallas_call(
        _count_body,
        grid=(NCHUNK,),
        in_specs=[
            pl.BlockSpec((1, 1, CHUNK), lambda i: (i, 0, 0),
                         memory_space=pltpu.SMEM),
            pl.BlockSpec((1, 1, CHUNK), lambda i: (i, 0, 0),
                         memory_space=pltpu.SMEM),
        ],
        out_specs=pl.BlockSpec((CNT_ROWS, HID), lambda i: (0, 0)),
        out_shape=jax.ShapeDtypeStruct((CNT_ROWS, HID), jnp.float32),
        scratch_shapes=[pltpu.VMEM((CNT_ROWS, HID), jnp.float32)
                        for _ in range(NACC)],
        compiler_params=pltpu.CompilerParams(
            dimension_semantics=("arbitrary",)),
    )(dst3, typ3)


def _make_layer_body(do_relu):
    def _layer_body(src_ref, dst_ref, typ_ref, tv_ref, x_ref, inv_ref,
                    wself_ref, wpart_ref, perm_ref, root_ref, bias_ref,
                    out_ref, xc_ref, sc2_ref, a0, a1, a2, a3):
        pid = pl.program_id(0)
        accs = (a0, a1, a2, a3)

        @pl.when(pid == 0)
        def _init():
            for a in accs:
                a[...] = jnp.zeros_like(a)

        lane_iota = jax.lax.broadcasted_iota(jnp.int32, (1, HID), 1)

        def gather_body(i, carry):
            for k in range(NACC):
                e = i * NACC + k
                s = src_ref[0, 0, e]
                d = dst_ref[0, 0, e]
                t = typ_ref[0, 0, e]
                idx = t * N_NODES + d
                xc_ref[pl.ds(e, 1), :] = x_ref[pl.ds(s, 1), :]
                row = inv_ref[pl.ds(jax.lax.shift_right_logical(idx, 7), 1), :]
                oh = (lane_iota
                      == jnp.bitwise_and(idx, 127)).astype(jnp.float32)
                sc2_ref[pl.ds(e, 1), :] = row * oh
            return carry

        jax.lax.fori_loop(0, CHUNK // NACC, gather_body, 0)

        sc = jnp.sum(sc2_ref[...], axis=1, keepdims=True)
        xs = xc_ref[...] * sc
        xw = jnp.dot(xs, perm_ref[...],
                     preferred_element_type=jnp.float32)
        tv = tv_ref[0]
        h = jnp.zeros_like(xs)
        for r in range(NREL):
            m = (tv == r).astype(jnp.float32)
            h += m * (xs * wself_ref[r, :][None, :]
                      + xw * wpart_ref[r, :][None, :])
        xc_ref[...] = h

        def scatter_body(i, carry):
            for k in range(NACC):
                e = i * NACC + k
                d = dst_ref[0, 0, e]
                accs[k][pl.ds(d, 1), :] += xc_ref[pl.ds(e, 1), :]
            return carry

        jax.lax.fori_loop(0, CHUNK // NACC, scatter_body, 0)

        @pl.when(pid == NCHUNK - 1)
        def _finish():
            res = (jnp.dot(x_ref[...], root_ref[...],
                           preferred_element_type=jnp.float32)
                   + bias_ref[...]
                   + a0[...] + a1[...] + a2[...] + a3[...])
            out_ref[...] = jnp.maximum(res, 0.0) if do_relu else res

    return _layer_body


def _layer(src3, dst3, typ3, tv3, x, inv, wself, wpart, perm, root, bias2d,
           do_relu):
    smem = lambda: pl.BlockSpec((1, 1, CHUNK), lambda i: (i, 0, 0),
                                memory_space=pltpu.SMEM)
    full = lambda shape: pl.BlockSpec(shape, lambda i: tuple(0 for _ in shape))
    return pl.pallas_call(
        _make_layer_body(do_relu),
        grid=(NCHUNK,),
        in_specs=[
            smem(), smem(), smem(),
            pl.BlockSpec((1, CHUNK, 1), lambda i: (i, 0, 0)),
            full((N_NODES, HID)),
            full((CNT_ROWS, HID)),
            full((NREL, HID)),
            full((NREL, HID)),
            full((HID, HID)),
            full((HID, HID)),
            full((1, HID)),
        ],
        out_specs=pl.BlockSpec((N_NODES, HID), lambda i: (0, 0)),
        out_shape=jax.ShapeDtypeStruct((N_NODES, HID), jnp.float32),
        scratch_shapes=[
            pltpu.VMEM((CHUNK, HID), jnp.float32),
            pltpu.VMEM((CHUNK, HID), jnp.float32),
        ] + [pltpu.VMEM((N_NODES, HID), jnp.float32) for _ in range(NACC)],
        compiler_params=pltpu.CompilerParams(
            dimension_semantics=("arbitrary",)),
    )(src3, dst3, typ3, tv3, x, inv, wself, wpart, perm, root, bias2d)


def _lane_weights(w):
    wself = jnp.einsum('rbii->rbi', w).reshape(NREL, HID)
    wpart = jnp.einsum('rbii->rbi', w[:, :, ::-1, :]).reshape(NREL, HID)
    return wself, wpart


def kernel(edge_index, edge_type, node_emb, weight1, root1, bias1,
           weight2, root2, bias2):
    src3 = edge_index[0].reshape(NCHUNK, 1, CHUNK)
    dst3 = edge_index[1].reshape(NCHUNK, 1, CHUNK)
    typ3 = edge_type.reshape(NCHUNK, 1, CHUNK)
    tv3 = edge_type.reshape(NCHUNK, CHUNK, 1)

    inv = _counts(dst3, typ3)

    lanes = jnp.arange(HID, dtype=jnp.int32)
    perm = (lanes[:, None] == (lanes[None, :] ^ 1)).astype(jnp.float32).T

    wself1, wpart1 = _lane_weights(weight1)
    wself2, wpart2 = _lane_weights(weight2)

    x1 = _layer(src3, dst3, typ3, tv3, node_emb, inv, wself1, wpart1, perm,
                root1, bias1.reshape(1, HID), do_relu=True)
    x2 = _layer(src3, dst3, typ3, tv3, x1, inv, wself2, wpart2, perm,
                root2, bias2.reshape(1, HID), do_relu=False)
    return x2


# speedup vs baseline: 6.5323x; 1.1868x over previous
import jax
import jax.numpy as jnp
from jax.experimental import pallas as pl
from jax.experimental.pallas import tpu as pltpu

N_NODES = 10000
HID = 128
NREL = 8
NEDGE = 320000
CHUNK = 512
NCHUNK = NEDGE // CHUNK


CNT_ROWS = NREL * N_NODES // HID


NACC = 8


def _count_body(dst_ref, typ_ref, inv_ref, *cnts):
    pid = pl.program_id(0)

    @pl.when(pid == 0)
    def _init():
        for c in cnts:
            c[...] = jnp.zeros_like(c)

    lane_iota = jax.lax.broadcasted_iota(jnp.int32, (1, HID), 1)

    def body(i, carry):
        for k in range(NACC):
            e = i * NACC + k
            d = dst_ref[0, 0, e]
            t = typ_ref[0, 0, e]
            idx = t * N_NODES + d
            row = jax.lax.shift_right_logical(idx, 7)
            oh = (lane_iota == jnp.bitwise_and(idx, 127)).astype(jnp.float32)
            cnts[k][pl.ds(row, 1), :] += oh
        return carry

    jax.lax.fori_loop(0, CHUNK // NACC, body, 0)

    @pl.when(pid == NCHUNK - 1)
    def _finish():
        tot = cnts[0][...]
        for c in cnts[1:]:
            tot = tot + c[...]
        inv_ref[...] = 1.0 / jnp.maximum(tot, 1.0)


def _counts(dst3, typ3):
    return pl.pallas_call(
        _count_body,
        grid=(NCHUNK,),
        in_specs=[
            pl.BlockSpec((1, 1, CHUNK), lambda i: (i, 0, 0),
                         memory_space=pltpu.SMEM),
            pl.BlockSpec((1, 1, CHUNK), lambda i: (i, 0, 0),
                         memory_space=pltpu.SMEM),
        ],
        out_specs=pl.BlockSpec((CNT_ROWS, HID), lambda i: (0, 0)),
        out_shape=jax.ShapeDtypeStruct((CNT_ROWS, HID), jnp.float32),
        scratch_shapes=[pltpu.VMEM((CNT_ROWS, HID), jnp.float32)
                        for _ in range(NACC)],
        compiler_params=pltpu.CompilerParams(
            dimension_semantics=("arbitrary",)),
    )(dst3, typ3)


def _make_layer_body(do_relu):
    def _layer_body(src_ref, dst_ref, typ_ref, tv_ref, x_ref, inv_ref,
                    wself_ref, wpart_ref, perm_ref, root_ref, bias_ref,
                    out_ref, xc_ref, sc2_ref, *accs):
        pid = pl.program_id(0)

        @pl.when(pid == 0)
        def _init():
            for a in accs:
                a[...] = jnp.zeros_like(a)

        lane_iota = jax.lax.broadcasted_iota(jnp.int32, (1, HID), 1)

        def gather_body(i, carry):
            for k in range(NACC):
                e = i * NACC + k
                s = src_ref[0, 0, e]
                d = dst_ref[0, 0, e]
                t = typ_ref[0, 0, e]
                idx = t * N_NODES + d
                xc_ref[pl.ds(e, 1), :] = x_ref[pl.ds(s, 1), :]
                row = inv_ref[pl.ds(jax.lax.shift_right_logical(idx, 7), 1), :]
                oh = (lane_iota
                      == jnp.bitwise_and(idx, 127)).astype(jnp.float32)
                sc2_ref[pl.ds(e, 1), :] = row * oh
            return carry

        jax.lax.fori_loop(0, CHUNK // NACC, gather_body, 0)

        sc = jnp.sum(sc2_ref[...], axis=1, keepdims=True)
        xs = xc_ref[...] * sc
        xw = jnp.dot(xs, perm_ref[...],
                     preferred_element_type=jnp.float32)
        tv = tv_ref[0]
        h = jnp.zeros_like(xs)
        for r in range(NREL):
            m = (tv == r).astype(jnp.float32)
            h += m * (xs * wself_ref[r, :][None, :]
                      + xw * wpart_ref[r, :][None, :])
        xc_ref[...] = h

        def scatter_body(i, carry):
            for k in range(NACC):
                e = i * NACC + k
                d = dst_ref[0, 0, e]
                accs[k][pl.ds(d, 1), :] += xc_ref[pl.ds(e, 1), :]
            return carry

        jax.lax.fori_loop(0, CHUNK // NACC, scatter_body, 0)

        @pl.when(pid == NCHUNK - 1)
        def _finish():
            res = (jnp.dot(x_ref[...], root_ref[...],
                           preferred_element_type=jnp.float32)
                   + bias_ref[...])
            for a in accs:
                res = res + a[...]
            out_ref[...] = jnp.maximum(res, 0.0) if do_relu else res

    return _layer_body


def _layer(src3, dst3, typ3, tv3, x, inv, wself, wpart, perm, root, bias2d,
           do_relu):
    smem = lambda: pl.BlockSpec((1, 1, CHUNK), lambda i: (i, 0, 0),
                                memory_space=pltpu.SMEM)
    full = lambda shape: pl.BlockSpec(shape, lambda i: tuple(0 for _ in shape))
    return pl.pallas_call(
        _make_layer_body(do_relu),
        grid=(NCHUNK,),
        in_specs=[
            smem(), smem(), smem(),
            pl.BlockSpec((1, CHUNK, 1), lambda i: (i, 0, 0)),
            full((N_NODES, HID)),
            full((CNT_ROWS, HID)),
            full((NREL, HID)),
            full((NREL, HID)),
            full((HID, HID)),
            full((HID, HID)),
            full((1, HID)),
        ],
        out_specs=pl.BlockSpec((N_NODES, HID), lambda i: (0, 0)),
        out_shape=jax.ShapeDtypeStruct((N_NODES, HID), jnp.float32),
        scratch_shapes=[
            pltpu.VMEM((CHUNK, HID), jnp.float32),
            pltpu.VMEM((CHUNK, HID), jnp.float32),
        ] + [pltpu.VMEM((N_NODES, HID), jnp.float32) for _ in range(NACC)],
        compiler_params=pltpu.CompilerParams(
            dimension_semantics=("arbitrary",)),
    )(src3, dst3, typ3, tv3, x, inv, wself, wpart, perm, root, bias2d)


def _lane_weights(w):
    wself = jnp.einsum('rbii->rbi', w).reshape(NREL, HID)
    wpart = jnp.einsum('rbii->rbi', w[:, :, ::-1, :]).reshape(NREL, HID)
    return wself, wpart


def kernel(edge_index, edge_type, node_emb, weight1, root1, bias1,
           weight2, root2, bias2):
    src3 = edge_index[0].reshape(NCHUNK, 1, CHUNK)
    dst3 = edge_index[1].reshape(NCHUNK, 1, CHUNK)
    typ3 = edge_type.reshape(NCHUNK, 1, CHUNK)
    tv3 = edge_type.reshape(NCHUNK, CHUNK, 1)

    inv = _counts(dst3, typ3)

    lanes = jnp.arange(HID, dtype=jnp.int32)
    perm = (lanes[:, None] == (lanes[None, :] ^ 1)).astype(jnp.float32).T

    wself1, wpart1 = _lane_weights(weight1)
    wself2, wpart2 = _lane_weights(weight2)

    x1 = _layer(src3, dst3, typ3, tv3, node_emb, inv, wself1, wpart1, perm,
                root1, bias1.reshape(1, HID), do_relu=True)
    x2 = _layer(src3, dst3, typ3, tv3, x1, inv, wself2, wpart2, perm,
                root2, bias2.reshape(1, HID), do_relu=False)
    return x2
